# baseline (speedup 1.0000x reference)
import functools

import jax
import jax.numpy as jnp
from jax import lax
from jax.experimental import pallas as pl
from jax.experimental.pallas import tpu as pltpu
from jax.experimental.pallas import tpu_sc as plsc

D = 64
V = 1_000_000
B = 16384
SEQ = 20
NC = 2
NS = 16
NW = NC * NS
G = 32
CHUNKS = B // (NW * G)
ROWS = G * SEQ
NIDX = ROWS // 128
IDXROWS = B * SEQ // 128
INV400 = 1.0 / (SEQ * SEQ)
NSTG = 6

VB = 256
VBLK = V // 128
NBLK = V // VB
VTAIL = V - NBLK * VB
NB = NBLK // NW
VUNROLL = 4


@functools.partial(
    pl.kernel,
    out_type=jax.ShapeDtypeStruct((V * D,), jnp.float32),
    mesh=plsc.VectorSubcoreMesh(core_axis_name="c", subcore_axis_name="s"),
    compiler_params=pltpu.CompilerParams(
        needs_layout_passes=False, disable_bounds_checks=True),
    scratch_types=[
        pltpu.VMEM((2, D, VB + 1), jnp.float32),
        pltpu.VMEM((2 * VB * D,), jnp.float32),
        pltpu.SemaphoreType.DMA,
        pltpu.SemaphoreType.DMA,
    ],
)
def _repack_sc(tt_hbm, tail_hbm, out_hbm, in_v, out_flat, sem_i, sem_o):
    wid = lax.axis_index("s") * NC + lax.axis_index("c")
    lane = lax.iota(jnp.int32, 16)
    dvecs = [dblk * 16 + lane for dblk in range(D // 16)]

    def blk(nb):
        return wid + nb * NW

    def fire_ins(nb):
        b = blk(nb)
        return [
            pltpu.make_async_copy(
                tt_hbm.at[pl.ds(r * 8, 8), pl.ds(b * VB, VB)],
                in_v.at[nb % 2, pl.ds(r * 8, 8), pl.ds(0, VB)], sem_i)
            for r in range(D // 8)
        ]

    def fire_out(nb):
        b = blk(nb)
        return pltpu.make_async_copy(
            out_flat.at[pl.ds((nb % 2) * (VB * D), VB * D)],
            out_hbm.at[pl.ds(b * (VB * D), VB * D)],
            sem_o)

    nblocks = jnp.where(wid < NBLK - NB * NW, NB + 1, NB)

    def start_ins(nb):
        for cp in fire_ins(nb):
            cp.start()

    def wait_ins(nb):
        for cp in fire_ins(nb):
            cp.wait()

    pl.when(nblocks > 0)(lambda: start_ins(0))

    def body(nb, carry):
        par = nb % 2
        wait_ins(nb)
        pl.when(nb + 1 < nblocks)(lambda: start_ins(nb + 1))
        pl.when(nb >= 2)(lambda: fire_out(nb - 2).wait())
        pbase = par * (VB * D)
        pv = jnp.full((16,), par, jnp.int32)

        @plsc.parallel_loop(0, VB // VUNROLL, unroll=4)
        def tr(i):
            v0 = i * VUNROLL
            for u in range(VUNROLL):
                v = v0 + u
                vv = jnp.full((16,), v, jnp.int32)
                for dblk in range(D // 16):
                    g = plsc.load_gather(in_v, [pv, dvecs[dblk], vv])
                    out_flat[pl.ds(pbase + v * D + dblk * 16, 16)] = g
        fire_out(nb).start()
        return carry

    lax.fori_loop(0, nblocks, body, 0)
    pl.when(nblocks >= 2)(lambda: fire_out(nblocks - 2).wait())
    pl.when(nblocks >= 1)(lambda: fire_out(nblocks - 1).wait())

    @pl.when(wid == NW - 1)
    def _tail():
        cp_in = pltpu.make_async_copy(
            tail_hbm, out_flat.at[pl.ds(0, VTAIL * D)], sem_i)
        cp_in.start()
        cp_in.wait()
        cp_out = pltpu.make_async_copy(
            out_flat.at[pl.ds(0, VTAIL * D)],
            out_hbm.at[pl.ds(VBLK * 128 * D, VTAIL * D)], sem_o)
        cp_out.start()
        cp_out.wait()


@functools.partial(
    pl.kernel,
    out_type=jax.ShapeDtypeStruct((B,), jnp.float32),
    mesh=plsc.VectorSubcoreMesh(core_axis_name="c", subcore_axis_name="s"),
    compiler_params=pltpu.CompilerParams(
        needs_layout_passes=False, use_tc_tiling_on_sc=False,
        disable_bounds_checks=True),
    scratch_types=[
        pltpu.VMEM((2, NIDX, 128), jnp.int32),
        pltpu.VMEM((2, ROWS, D), jnp.float32),
        pltpu.VMEM((3, G, D), jnp.float32),
        pltpu.VMEM((CHUNKS * G,), jnp.float32),
        pltpu.SemaphoreType.DMA,
        pltpu.SemaphoreType.DMA,
    ],
)
def _feel_sc(table_hbm, q_hbm, p_hbm, n_hbm, out_hbm,
             idx_v, rows_v, acc_v, out_v, sem_g, sem_i):
    wid = lax.axis_index("s") * NC + lax.axis_index("c")
    idx_base = wid * (CHUNKS * NIDX)
    idx_refs = [q_hbm, p_hbm, n_hbm]

    def idx_src(s, cc):
        chunk = cc * 2 + (0 if s < 3 else 1)
        return idx_refs[s % 3].at[pl.ds(idx_base + chunk * NIDX, NIDX)]

    def fire_idx(s, cc):
        return pltpu.async_copy(idx_src(s % NSTG, cc), idx_v.at[(s % NSTG) % 2],
                                sem_i)

    def fire_gathers(s, cc):
        par = s % 2
        return [
            pltpu.async_copy(
                table_hbm.at[idx_v.at[par, j]],
                rows_v.at[par, pl.ds(j * 128, 128)],
                sem_g,
            )
            for j in range(NIDX)
        ]

    fire_idx(0, 0).wait()
    fire_gathers(0, 0)
    fire_idx(1, 0)

    def dchunk(cc, carry):
        for s in range(NSTG):
            k = s % 3
            chunk = cc * 2 + (0 if s < 3 else 1)
            par = s % 2

            for j in range(NIDX):
                pltpu.make_async_copy(
                    table_hbm.at[idx_v.at[par, j]],
                    rows_v.at[par, pl.ds(j * 128, 128)],
                    sem_g,
                ).wait()

            def _issue_next():
                nxt = s + 1
                ncc = cc + (1 if nxt >= NSTG else 0)
                pltpu.make_async_copy(idx_src(nxt % NSTG, ncc),
                                      idx_v.at[(nxt % NSTG) % 2],
                                      sem_i).wait()
                fire_gathers(nxt, ncc)

            def _issue_idx2():
                n2 = s + 2
                ncc2 = cc + (1 if n2 >= NSTG else 0)
                fire_idx(n2, ncc2)

            if s == NSTG - 1:
                pl.when(cc < CHUNKS // 2 - 1)(_issue_next)
            else:
                _issue_next()
            if s >= NSTG - 2:
                pl.when(cc < CHUNKS // 2 - 1)(_issue_idx2)
            else:
                _issue_idx2()

            @plsc.parallel_loop(0, G, unroll=2)
            def acc_body(e):
                for blk in range(D // 16):
                    sl = pl.ds(blk * 16, 16)
                    vs = [rows_v[par, e * SEQ + q, sl] for q in range(SEQ)]
                    while len(vs) > 1:
                        nxt_vs = [a + b for a, b in zip(vs[0::2], vs[1::2])]
                        if len(vs) % 2:
                            nxt_vs[-1] = nxt_vs[-1] + vs[-1]
                        vs = nxt_vs
                    acc_v[k, e, sl] = vs[0]

            if k == 2:
                lane = lax.iota(jnp.int32, 16)
                tbl = [jnp.full((16,), i, jnp.int32) for i in range(3)]
                for h in range(G // 16):
                    e_idx = lane + (h * 16)
                    d = jnp.zeros((16,), jnp.float32)
                    for dim in range(D):
                        dimv = jnp.full((16,), dim, jnp.int32)
                        qv = plsc.load_gather(acc_v, [tbl[0], e_idx, dimv])
                        pv = plsc.load_gather(acc_v, [tbl[1], e_idx, dimv])
                        nv = plsc.load_gather(acc_v, [tbl[2], e_idx, dimv])
                        d = d + qv * (pv - nv)
                    out_v[pl.ds(chunk * G + h * 16, 16)] = jnp.maximum(
                        0.0, 1.0 - d * INV400)
        return carry

    lax.fori_loop(0, CHUNKS // 2, dchunk, 0)
    pltpu.sync_copy(out_v, out_hbm.at[pl.ds(wid * (CHUNKS * G), CHUNKS * G)])


def kernel(query, pos, neg, table):
    q = query.astype(jnp.int32).reshape(IDXROWS, 128)
    p = pos.astype(jnp.int32).reshape(IDXROWS, 128)
    n = neg.astype(jnp.int32).reshape(IDXROWS, 128)
    tail = table[VBLK * 128:, :].reshape(VTAIL * D)
    table_lin = _repack_sc(table.T, tail).reshape(V, D)
    return _feel_sc(table_lin, q, p, n)


# speedup vs baseline: 1.2892x; 1.2892x over previous
import functools

import jax
import jax.numpy as jnp
from jax import lax
from jax.experimental import pallas as pl
from jax.experimental.pallas import tpu as pltpu
from jax.experimental.pallas import tpu_sc as plsc

D = 64
DP = 128
V = 1_000_000
B = 16384
SEQ = 20
NC = 2
NS = 16
NW = NC * NS
G = 16
CHUNKS = B // (NW * G)
ROWS = G * SEQ
IB = 64
NIDX = ROWS // IB
IDXROWS = B * SEQ // IB
INV400 = 1.0 / (SEQ * SEQ)
NSTG = 6


@functools.partial(
    pl.kernel,
    out_type=jax.ShapeDtypeStruct((B,), jnp.float32),
    mesh=plsc.VectorSubcoreMesh(core_axis_name="c", subcore_axis_name="s"),
    compiler_params=pltpu.CompilerParams(
        needs_layout_passes=False, use_tc_tiling_on_sc=False,
        disable_bounds_checks=True),
    scratch_types=[
        pltpu.VMEM((2, NIDX, IB), jnp.int32),
        pltpu.VMEM((2, ROWS, DP), jnp.float32),
        pltpu.VMEM((3, G, D), jnp.float32),
        pltpu.VMEM((CHUNKS * G,), jnp.float32),
        pltpu.SemaphoreType.DMA,
        pltpu.SemaphoreType.DMA,
    ],
)
def _feel_sc(table_hbm, q_hbm, p_hbm, n_hbm, out_hbm,
             idx_v, rows_v, acc_v, out_v, sem_g, sem_i):
    wid = lax.axis_index("s") * NC + lax.axis_index("c")
    idx_base = wid * (CHUNKS * NIDX)
    idx_refs = [q_hbm, p_hbm, n_hbm]

    def idx_src(s, cc):
        chunk = cc * 2 + (0 if s < 3 else 1)
        return idx_refs[s % 3].at[pl.ds(idx_base + chunk * NIDX, NIDX)]

    def fire_idx(s, cc):
        return pltpu.async_copy(idx_src(s % NSTG, cc), idx_v.at[(s % NSTG) % 2],
                                sem_i)

    def fire_gathers(s, cc):
        par = s % 2
        return [
            pltpu.async_copy(
                table_hbm.at[idx_v.at[par, j]],
                rows_v.at[par, pl.ds(j * IB, IB)],
                sem_g,
            )
            for j in range(NIDX)
        ]

    fire_idx(0, 0).wait()
    fire_gathers(0, 0)
    fire_idx(1, 0)

    def dchunk(cc, carry):
        for s in range(NSTG):
            k = s % 3
            chunk = cc * 2 + (0 if s < 3 else 1)
            par = s % 2

            for j in range(NIDX):
                pltpu.make_async_copy(
                    table_hbm.at[idx_v.at[par, j]],
                    rows_v.at[par, pl.ds(j * IB, IB)],
                    sem_g,
                ).wait()

            def _issue_next():
                nxt = s + 1
                ncc = cc + (1 if nxt >= NSTG else 0)
                pltpu.make_async_copy(idx_src(nxt % NSTG, ncc),
                                      idx_v.at[(nxt % NSTG) % 2],
                                      sem_i).wait()
                fire_gathers(nxt, ncc)

            def _issue_idx2():
                n2 = s + 2
                ncc2 = cc + (1 if n2 >= NSTG else 0)
                fire_idx(n2, ncc2)

            if s == NSTG - 1:
                pl.when(cc < CHUNKS // 2 - 1)(_issue_next)
            else:
                _issue_next()
            if s >= NSTG - 2:
                pl.when(cc < CHUNKS // 2 - 1)(_issue_idx2)
            else:
                _issue_idx2()

            @plsc.parallel_loop(0, G, unroll=2)
            def acc_body(e):
                for blk in range(D // 16):
                    sl = pl.ds(blk * 16, 16)
                    vs = [rows_v[par, e * SEQ + q, sl] for q in range(SEQ)]
                    while len(vs) > 1:
                        nxt_vs = [a + b for a, b in zip(vs[0::2], vs[1::2])]
                        if len(vs) % 2:
                            nxt_vs[-1] = nxt_vs[-1] + vs[-1]
                        vs = nxt_vs
                    acc_v[k, e, sl] = vs[0]

            if k == 2:
                lane = lax.iota(jnp.int32, 16)
                tbl = [jnp.full((16,), i, jnp.int32) for i in range(3)]
                d = jnp.zeros((16,), jnp.float32)
                for dim in range(D):
                    dimv = jnp.full((16,), dim, jnp.int32)
                    qv = plsc.load_gather(acc_v, [tbl[0], lane, dimv])
                    pv = plsc.load_gather(acc_v, [tbl[1], lane, dimv])
                    nv = plsc.load_gather(acc_v, [tbl[2], lane, dimv])
                    d = d + qv * (pv - nv)
                out_v[pl.ds(chunk * G, 16)] = jnp.maximum(
                    0.0, 1.0 - d * INV400)
        return carry

    lax.fori_loop(0, CHUNKS // 2, dchunk, 0)
    pltpu.sync_copy(out_v, out_hbm.at[pl.ds(wid * (CHUNKS * G), CHUNKS * G)])


def kernel(query, pos, neg, table):
    q = query.astype(jnp.int32).reshape(IDXROWS, IB)
    p = pos.astype(jnp.int32).reshape(IDXROWS, IB)
    n = neg.astype(jnp.int32).reshape(IDXROWS, IB)
    table_p = jnp.pad(table, ((0, 0), (0, DP - D)))
    return _feel_sc(table_p, q, p, n)


# speedup vs baseline: 1.3852x; 1.0745x over previous
import functools

import jax
import jax.numpy as jnp
from jax import lax
from jax.experimental import pallas as pl
from jax.experimental.pallas import tpu as pltpu
from jax.experimental.pallas import tpu_sc as plsc

D = 64
DP = 128
V = 1_000_000
B = 16384
SEQ = 20
NC = 2
NS = 16
NW = NC * NS
G = 16
CHUNKS = B // (NW * G)
ROWS = G * SEQ
IB = 64
NIDX = ROWS // IB
IDXROWS = B * SEQ // IB
INV400 = 1.0 / (SEQ * SEQ)
NSTG = 6


@functools.partial(
    pl.kernel,
    out_type=jax.ShapeDtypeStruct((B,), jnp.float32),
    mesh=plsc.VectorSubcoreMesh(core_axis_name="c", subcore_axis_name="s"),
    compiler_params=pltpu.CompilerParams(
        needs_layout_passes=False, use_tc_tiling_on_sc=False,
        disable_bounds_checks=True),
    scratch_types=[
        pltpu.VMEM((2, NIDX, IB), jnp.int32),
        pltpu.VMEM((2, ROWS, D), jnp.float32),
        pltpu.VMEM((3, G, D), jnp.float32),
        pltpu.VMEM((CHUNKS * G,), jnp.float32),
        pltpu.SemaphoreType.DMA,
        pltpu.SemaphoreType.DMA,
    ],
)
def _feel_sc(table_hbm, q_hbm, p_hbm, n_hbm, out_hbm,
             idx_v, rows_v, acc_v, out_v, sem_g, sem_i):
    wid = lax.axis_index("s") * NC + lax.axis_index("c")
    idx_base = wid * (CHUNKS * NIDX)
    idx_refs = [q_hbm, p_hbm, n_hbm]

    def idx_src(s, cc):
        chunk = cc * 2 + (0 if s < 3 else 1)
        return idx_refs[s % 3].at[pl.ds(idx_base + chunk * NIDX, NIDX)]

    def fire_idx(s, cc):
        return pltpu.async_copy(idx_src(s % NSTG, cc), idx_v.at[(s % NSTG) % 2],
                                sem_i)

    def fire_gathers(s, cc):
        par = s % 2
        return [
            pltpu.async_copy(
                table_hbm.at[idx_v.at[par, j]],
                rows_v.at[par, pl.ds(j * IB, IB)],
                sem_g,
            )
            for j in range(NIDX)
        ]

    fire_idx(0, 0).wait()
    fire_gathers(0, 0)
    fire_idx(1, 0)

    def dchunk(cc, carry):
        for s in range(NSTG):
            k = s % 3
            chunk = cc * 2 + (0 if s < 3 else 1)
            par = s % 2

            for j in range(NIDX):
                pltpu.make_async_copy(
                    table_hbm.at[idx_v.at[par, j]],
                    rows_v.at[par, pl.ds(j * IB, IB)],
                    sem_g,
                ).wait()

            def _issue_next():
                nxt = s + 1
                ncc = cc + (1 if nxt >= NSTG else 0)
                pltpu.make_async_copy(idx_src(nxt % NSTG, ncc),
                                      idx_v.at[(nxt % NSTG) % 2],
                                      sem_i).wait()
                fire_gathers(nxt, ncc)

            def _issue_idx2():
                n2 = s + 2
                ncc2 = cc + (1 if n2 >= NSTG else 0)
                fire_idx(n2, ncc2)

            if s == NSTG - 1:
                pl.when(cc < CHUNKS // 2 - 1)(_issue_next)
            else:
                _issue_next()
            if s >= NSTG - 2:
                pl.when(cc < CHUNKS // 2 - 1)(_issue_idx2)
            else:
                _issue_idx2()

            @plsc.parallel_loop(0, G, unroll=2)
            def acc_body(e):
                for blk in range(D // 16):
                    sl = pl.ds(blk * 16, 16)
                    vs = [rows_v[par, e * SEQ + q, sl] for q in range(SEQ)]
                    while len(vs) > 1:
                        nxt_vs = [a + b for a, b in zip(vs[0::2], vs[1::2])]
                        if len(vs) % 2:
                            nxt_vs[-1] = nxt_vs[-1] + vs[-1]
                        vs = nxt_vs
                    acc_v[k, e, sl] = vs[0]

            if k == 2:
                lane = lax.iota(jnp.int32, 16)
                tbl = [jnp.full((16,), i, jnp.int32) for i in range(3)]
                d = jnp.zeros((16,), jnp.float32)
                for dim in range(D):
                    dimv = jnp.full((16,), dim, jnp.int32)
                    qv = plsc.load_gather(acc_v, [tbl[0], lane, dimv])
                    pv = plsc.load_gather(acc_v, [tbl[1], lane, dimv])
                    nv = plsc.load_gather(acc_v, [tbl[2], lane, dimv])
                    d = d + qv * (pv - nv)
                out_v[pl.ds(chunk * G, 16)] = jnp.maximum(
                    0.0, 1.0 - d * INV400)
        return carry

    lax.fori_loop(0, CHUNKS // 2, dchunk, 0)
    pltpu.sync_copy(out_v, out_hbm.at[pl.ds(wid * (CHUNKS * G), CHUNKS * G)])


def kernel(query, pos, neg, table):
    q = (query.astype(jnp.int32) * 2).reshape(IDXROWS, IB)
    p = (pos.astype(jnp.int32) * 2).reshape(IDXROWS, IB)
    n = (neg.astype(jnp.int32) * 2).reshape(IDXROWS, IB)
    table_p = jnp.pad(table, ((0, 0), (0, DP - D))).reshape(2 * V, D)
    return _feel_sc(table_p, q, p, n)


# speedup vs baseline: 1.4410x; 1.0403x over previous
import functools

import jax
import jax.numpy as jnp
from jax import lax
from jax.experimental import pallas as pl
from jax.experimental.pallas import tpu as pltpu
from jax.experimental.pallas import tpu_sc as plsc

D = 64
DP = 128
V = 1_000_000
B = 16384
SEQ = 20
NC = 2
NS = 16
NW = NC * NS
G = 32
CHUNKS = B // (NW * G)
ROWS = G * SEQ
IB = 128
NIDX = ROWS // IB
IDXROWS = B * SEQ // IB
INV400 = 1.0 / (SEQ * SEQ)
NSTG = 6


@functools.partial(
    pl.kernel,
    out_type=jax.ShapeDtypeStruct((B,), jnp.float32),
    mesh=plsc.VectorSubcoreMesh(core_axis_name="c", subcore_axis_name="s"),
    compiler_params=pltpu.CompilerParams(
        needs_layout_passes=False, use_tc_tiling_on_sc=False,
        disable_bounds_checks=True),
    scratch_types=[
        pltpu.VMEM((2, NIDX, IB), jnp.int32),
        pltpu.VMEM((2, ROWS, D), jnp.float32),
        pltpu.VMEM((3, G, D), jnp.float32),
        pltpu.VMEM((CHUNKS * G,), jnp.float32),
        pltpu.SemaphoreType.DMA,
        pltpu.SemaphoreType.DMA,
    ],
)
def _feel_sc(table_hbm, q_hbm, p_hbm, n_hbm, out_hbm,
             idx_v, rows_v, acc_v, out_v, sem_g, sem_i):
    wid = lax.axis_index("s") * NC + lax.axis_index("c")
    idx_base = wid * (CHUNKS * NIDX)
    idx_refs = [q_hbm, p_hbm, n_hbm]

    def idx_src(s, cc):
        chunk = cc * 2 + (0 if s < 3 else 1)
        return idx_refs[s % 3].at[pl.ds(idx_base + chunk * NIDX, NIDX)]

    def fire_idx(s, cc):
        return pltpu.async_copy(idx_src(s % NSTG, cc), idx_v.at[(s % NSTG) % 2],
                                sem_i)

    def fire_gathers(s, cc):
        par = s % 2
        return [
            pltpu.async_copy(
                table_hbm.at[idx_v.at[par, j]],
                rows_v.at[par, pl.ds(j * IB, IB)],
                sem_g,
            )
            for j in range(NIDX)
        ]

    fire_idx(0, 0).wait()
    fire_gathers(0, 0)
    fire_idx(1, 0)

    def dchunk(cc, carry):
        for s in range(NSTG):
            k = s % 3
            chunk = cc * 2 + (0 if s < 3 else 1)
            par = s % 2

            for j in range(NIDX):
                pltpu.make_async_copy(
                    table_hbm.at[idx_v.at[par, j]],
                    rows_v.at[par, pl.ds(j * IB, IB)],
                    sem_g,
                ).wait()

            def _issue_next():
                nxt = s + 1
                ncc = cc + (1 if nxt >= NSTG else 0)
                pltpu.make_async_copy(idx_src(nxt % NSTG, ncc),
                                      idx_v.at[(nxt % NSTG) % 2],
                                      sem_i).wait()
                fire_gathers(nxt, ncc)

            def _issue_idx2():
                n2 = s + 2
                ncc2 = cc + (1 if n2 >= NSTG else 0)
                fire_idx(n2, ncc2)

            if s == NSTG - 1:
                pl.when(cc < CHUNKS // 2 - 1)(_issue_next)
            else:
                _issue_next()
            if s >= NSTG - 2:
                pl.when(cc < CHUNKS // 2 - 1)(_issue_idx2)
            else:
                _issue_idx2()

            @plsc.parallel_loop(0, G, unroll=2)
            def acc_body(e):
                for blk in range(D // 16):
                    sl = pl.ds(blk * 16, 16)
                    vs = [rows_v[par, e * SEQ + q, sl] for q in range(SEQ)]
                    while len(vs) > 1:
                        nxt_vs = [a + b for a, b in zip(vs[0::2], vs[1::2])]
                        if len(vs) % 2:
                            nxt_vs[-1] = nxt_vs[-1] + vs[-1]
                        vs = nxt_vs
                    acc_v[k, e, sl] = vs[0]

            if k == 2:
                lane = lax.iota(jnp.int32, 16)
                tbl = [jnp.full((16,), i, jnp.int32) for i in range(3)]
                for h in range(G // 16):
                    e_idx = lane + (h * 16)
                    d = jnp.zeros((16,), jnp.float32)
                    for dim in range(D):
                        dimv = jnp.full((16,), dim, jnp.int32)
                        qv = plsc.load_gather(acc_v, [tbl[0], e_idx, dimv])
                        pv = plsc.load_gather(acc_v, [tbl[1], e_idx, dimv])
                        nv = plsc.load_gather(acc_v, [tbl[2], e_idx, dimv])
                        d = d + qv * (pv - nv)
                    out_v[pl.ds(chunk * G + h * 16, 16)] = jnp.maximum(
                        0.0, 1.0 - d * INV400)
        return carry

    lax.fori_loop(0, CHUNKS // 2, dchunk, 0)
    pltpu.sync_copy(out_v, out_hbm.at[pl.ds(wid * (CHUNKS * G), CHUNKS * G)])


def kernel(query, pos, neg, table):
    q = (query.astype(jnp.int32) * 2).reshape(IDXROWS, IB)
    p = (pos.astype(jnp.int32) * 2).reshape(IDXROWS, IB)
    n = (neg.astype(jnp.int32) * 2).reshape(IDXROWS, IB)
    table_p = jnp.pad(table, ((0, 0), (0, DP - D))).reshape(2 * V, D)
    return _feel_sc(table_p, q, p, n)
